# Initial kernel scaffold; baseline (speedup 1.0000x reference)
import jax
import jax.numpy as jnp
from jax.experimental import pallas as pl


def kernel(hidden_states, Wr, Wk, Wv, Wo, w_decay, u_bonus, router_W):
    raise NotImplementedError("write your pallas kernel here")


# speedup vs baseline: 8.9612x; 8.9612x over previous
import functools

import jax
import jax.numpy as jnp
from jax.experimental import pallas as pl
from jax.experimental.pallas import tpu as pltpu


def _mm_body(x_ref, w_ref, o_ref, *, act):
    y = jnp.dot(x_ref[...], w_ref[...], preferred_element_type=jnp.float32)
    if act == "sigmoid":
        y = jax.nn.sigmoid(y)
    o_ref[...] = y


def _matmul(x, w, act=None, bn=256):
    n, d = x.shape
    d2 = w.shape[1]
    bn = min(bn, n)
    return pl.pallas_call(
        functools.partial(_mm_body, act=act),
        grid=(n // bn,),
        in_specs=[
            pl.BlockSpec((bn, d), lambda i: (i, 0)),
            pl.BlockSpec((d, d2), lambda i: (0, 0)),
        ],
        out_specs=pl.BlockSpec((bn, d2), lambda i: (i, 0)),
        out_shape=jax.ShapeDtypeStruct((n, d2), jnp.float32),
        compiler_params=pltpu.CompilerParams(
            dimension_semantics=("arbitrary",)),
    )(x, w)


def _scan_body(w_ref, u_ref, k_ref, v_ref, o_ref, a_ref, b_ref, *, sc):
    s = pl.program_id(0)

    @pl.when(s == 0)
    def _():
        a_ref[...] = jnp.zeros_like(a_ref)
        b_ref[...] = jnp.zeros_like(b_ref)

    decay = jnp.exp(-w_ref[...])
    ub = u_ref[...]

    def step(j, _):
        kt = k_ref[:, j]
        vt = v_ref[:, j]
        ek = jnp.exp(jnp.clip(kt, -30.0, 30.0))
        eku = jnp.exp(jnp.clip(ub + kt, -30.0, 30.0))
        a = a_ref[...]
        b = b_ref[...]
        o_ref[:, j] = (a + eku * vt) / (b + eku + 1e-8)
        a_ref[...] = decay * (a + ek * vt)
        b_ref[...] = decay * (b + ek)
        return 0

    jax.lax.fori_loop(0, sc, step, 0, unroll=2)


def _wkv_scan(k4, v4, wd, ub, sc=128):
    b, s, g, lanes = k4.shape
    sc = min(sc, s)
    return pl.pallas_call(
        functools.partial(_scan_body, sc=sc),
        grid=(s // sc,),
        in_specs=[
            pl.BlockSpec((g, lanes), lambda i: (0, 0)),
            pl.BlockSpec((g, lanes), lambda i: (0, 0)),
            pl.BlockSpec((b, sc, g, lanes), lambda i: (0, i, 0, 0)),
            pl.BlockSpec((b, sc, g, lanes), lambda i: (0, i, 0, 0)),
        ],
        out_specs=pl.BlockSpec((b, sc, g, lanes), lambda i: (0, i, 0, 0)),
        out_shape=jax.ShapeDtypeStruct((b, s, g, lanes), jnp.float32),
        scratch_shapes=[
            pltpu.VMEM((b, g, lanes), jnp.float32),
            pltpu.VMEM((b, g, lanes), jnp.float32),
        ],
        compiler_params=pltpu.CompilerParams(
            dimension_semantics=("arbitrary",)),
    )(wd, ub, k4, v4)


def _router_body(r_ref, wkv_ref, wo_ref, wt_ref,
                 probs_ref, idx_ref, wts_ref, cnt_ref, loss_ref,
                 acc_ref, *, bn, n_total, e, topk):
    i = pl.program_id(0)

    @pl.when(i == 0)
    def _():
        acc_ref[...] = jnp.zeros_like(acc_ref)
        cnt_ref[...] = jnp.zeros_like(cnt_ref)

    o = jnp.dot(r_ref[...] * wkv_ref[...], wo_ref[...],
                preferred_element_type=jnp.float32)
    logits = jnp.dot(o, wt_ref[...], preferred_element_type=jnp.float32)

    m = jnp.max(logits, axis=1, keepdims=True)
    p = jnp.exp(logits - m)
    probs = p / jnp.sum(p, axis=1, keepdims=True)
    probs_ref[...] = probs
    acc_ref[...] += jnp.sum(probs, axis=0, keepdims=True)

    iota = jax.lax.broadcasted_iota(jnp.int32, (bn, e), 1)
    work = probs
    vals, idxs = [], []
    cnt = jnp.zeros((1, e), jnp.int32)
    for _ in range(topk):
        mv = jnp.max(work, axis=1, keepdims=True)
        cand = jnp.where(work == mv, iota, e)
        mi = jnp.min(cand, axis=1, keepdims=True)
        sel = iota == mi
        vals.append(mv)
        idxs.append(mi)
        cnt = cnt + jnp.sum(sel.astype(jnp.int32), axis=0, keepdims=True)
        work = jnp.where(sel, -1.0, work)
    cnt_ref[...] += cnt

    v8 = jnp.concatenate(vals, axis=1)
    idx_ref[...] = jnp.concatenate(idxs, axis=1).astype(jnp.int32)
    wts_ref[...] = v8 / (jnp.sum(v8, axis=1, keepdims=True) + 1e-8)

    @pl.when(i == pl.num_programs(0) - 1)
    def _():
        mean = acc_ref[...] / float(n_total)
        u = 1.0 / e
        kl = jnp.sum(u * (jnp.log(u) - jnp.log(mean + 1e-20)))
        loss_ref[...] = jnp.full((1, 1), 1.0 / e) * kl


def _router(r, wkv, wo, wt, topk, bn=256):
    n, d = r.shape
    e = wt.shape[1]
    bn = min(bn, n)
    return pl.pallas_call(
        functools.partial(_router_body, bn=bn, n_total=n, e=e, topk=topk),
        grid=(n // bn,),
        in_specs=[
            pl.BlockSpec((bn, d), lambda i: (i, 0)),
            pl.BlockSpec((bn, d), lambda i: (i, 0)),
            pl.BlockSpec((d, d), lambda i: (0, 0)),
            pl.BlockSpec((d, e), lambda i: (0, 0)),
        ],
        out_specs=[
            pl.BlockSpec((bn, e), lambda i: (i, 0)),
            pl.BlockSpec((bn, topk), lambda i: (i, 0)),
            pl.BlockSpec((bn, topk), lambda i: (i, 0)),
            pl.BlockSpec((1, e), lambda i: (0, 0)),
            pl.BlockSpec((1, 1), lambda i: (0, 0)),
        ],
        out_shape=[
            jax.ShapeDtypeStruct((n, e), jnp.float32),
            jax.ShapeDtypeStruct((n, topk), jnp.int32),
            jax.ShapeDtypeStruct((n, topk), jnp.float32),
            jax.ShapeDtypeStruct((1, e), jnp.int32),
            jax.ShapeDtypeStruct((1, 1), jnp.float32),
        ],
        scratch_shapes=[pltpu.VMEM((1, e), jnp.float32)],
        compiler_params=pltpu.CompilerParams(
            dimension_semantics=("arbitrary",)),
    )(r, wkv, wo, wt)


def kernel(hidden_states, Wr, Wk, Wv, Wo, w_decay, u_bonus, router_W):
    b, s, d = hidden_states.shape
    e = router_W.shape[0]
    topk = 8
    n = b * s
    g = d // 128

    x = hidden_states.reshape(n, d)
    r = _matmul(x, Wr, act="sigmoid")
    k = _matmul(x, Wk)
    v = _matmul(x, Wv)

    k4 = k.reshape(b, s, g, 128)
    v4 = v.reshape(b, s, g, 128)
    wkv4 = _wkv_scan(k4, v4, w_decay.reshape(g, 128), u_bonus.reshape(g, 128))
    wkv = wkv4.reshape(n, d)

    probs, idx, wts, cnt, loss = _router(r, wkv, Wo, router_W.T, topk)
    return (idx.reshape(b, s, topk),
            wts.reshape(b, s, topk),
            loss.reshape(()),
            probs.reshape(b, s, e),
            cnt.reshape(e))


# speedup vs baseline: 9.4641x; 1.0561x over previous
import functools

import jax
import jax.numpy as jnp
from jax.experimental import pallas as pl
from jax.experimental.pallas import tpu as pltpu


def _mm_body(x_ref, w_ref, o_ref, *, act):
    y = jnp.dot(x_ref[...], w_ref[...], preferred_element_type=jnp.float32)
    if act == "sigmoid":
        y = jax.nn.sigmoid(y)
    o_ref[...] = y


def _matmul(x, w, act=None, bn=256):
    n, d = x.shape
    d2 = w.shape[1]
    bn = min(bn, n)
    return pl.pallas_call(
        functools.partial(_mm_body, act=act),
        grid=(n // bn,),
        in_specs=[
            pl.BlockSpec((bn, d), lambda i: (i, 0)),
            pl.BlockSpec((d, d2), lambda i: (0, 0)),
        ],
        out_specs=pl.BlockSpec((bn, d2), lambda i: (i, 0)),
        out_shape=jax.ShapeDtypeStruct((n, d2), jnp.float32),
        compiler_params=pltpu.CompilerParams(
            dimension_semantics=("arbitrary",)),
    )(x, w)


def _scan_body(w_ref, u_ref, k_ref, v_ref, o_ref, a_ref, b_ref, *, sc):
    s = pl.program_id(0)

    @pl.when(s == 0)
    def _():
        a_ref[...] = jnp.zeros_like(a_ref)
        b_ref[...] = jnp.zeros_like(b_ref)

    decay = jnp.exp(-w_ref[...])
    ub = u_ref[...]

    def step(j, _):
        kt = k_ref[:, j]
        vt = v_ref[:, j]
        ek = jnp.exp(jnp.clip(kt, -30.0, 30.0))
        eku = jnp.exp(jnp.clip(ub + kt, -30.0, 30.0))
        a = a_ref[...]
        b = b_ref[...]
        o_ref[:, j] = (a + eku * vt) / (b + eku + 1e-8)
        a_ref[...] = decay * (a + ek * vt)
        b_ref[...] = decay * (b + ek)
        return 0

    jax.lax.fori_loop(0, sc, step, 0, unroll=2)


def _wkv_scan(k3, v3, wd, ub, sc=128):
    b, s, d = k3.shape
    sc = min(sc, s)
    return pl.pallas_call(
        functools.partial(_scan_body, sc=sc),
        grid=(s // sc,),
        in_specs=[
            pl.BlockSpec((1, d), lambda i: (0, 0)),
            pl.BlockSpec((1, d), lambda i: (0, 0)),
            pl.BlockSpec((b, sc, d), lambda i: (0, i, 0)),
            pl.BlockSpec((b, sc, d), lambda i: (0, i, 0)),
        ],
        out_specs=pl.BlockSpec((b, sc, d), lambda i: (0, i, 0)),
        out_shape=jax.ShapeDtypeStruct((b, s, d), jnp.float32),
        scratch_shapes=[
            pltpu.VMEM((b, d), jnp.float32),
            pltpu.VMEM((b, d), jnp.float32),
        ],
        compiler_params=pltpu.CompilerParams(
            dimension_semantics=("arbitrary",)),
    )(wd, ub, k3, v3)


def _router_body(r_ref, wkv_ref, wo_ref, wt_ref,
                 probs_ref, idx_ref, wts_ref, cnt_ref, loss_ref,
                 acc_ref, *, bn, n_total, e, topk):
    i = pl.program_id(0)

    @pl.when(i == 0)
    def _():
        acc_ref[...] = jnp.zeros_like(acc_ref)
        cnt_ref[...] = jnp.zeros_like(cnt_ref)

    o = jnp.dot(r_ref[...] * wkv_ref[...], wo_ref[...],
                preferred_element_type=jnp.float32)
    logits = jnp.dot(o, wt_ref[...], preferred_element_type=jnp.float32)

    m = jnp.max(logits, axis=1, keepdims=True)
    p = jnp.exp(logits - m)
    probs = p / jnp.sum(p, axis=1, keepdims=True)
    probs_ref[...] = probs
    acc_ref[...] += jnp.sum(probs, axis=0, keepdims=True)

    iota = jax.lax.broadcasted_iota(jnp.int32, (bn, e), 1)
    work = probs
    vals, idxs = [], []
    cnt = jnp.zeros((1, e), jnp.int32)
    for _ in range(topk):
        mv = jnp.max(work, axis=1, keepdims=True)
        cand = jnp.where(work == mv, iota, e)
        mi = jnp.min(cand, axis=1, keepdims=True)
        sel = iota == mi
        vals.append(mv)
        idxs.append(mi)
        cnt = cnt + jnp.sum(sel.astype(jnp.int32), axis=0, keepdims=True)
        work = jnp.where(sel, -1.0, work)
    cnt_ref[...] += cnt

    v8 = jnp.concatenate(vals, axis=1)
    idx_ref[...] = jnp.concatenate(idxs, axis=1).astype(jnp.int32)
    wts_ref[...] = v8 / (jnp.sum(v8, axis=1, keepdims=True) + 1e-8)

    @pl.when(i == pl.num_programs(0) - 1)
    def _():
        mean = acc_ref[...] / float(n_total)
        u = 1.0 / e
        kl = jnp.sum(u * (jnp.log(u) - jnp.log(mean + 1e-20)))
        loss_ref[...] = jnp.full((1, 1), 1.0 / e) * kl


def _router(r, wkv, wo, wt, topk, bn=256):
    n, d = r.shape
    e = wt.shape[1]
    bn = min(bn, n)
    return pl.pallas_call(
        functools.partial(_router_body, bn=bn, n_total=n, e=e, topk=topk),
        grid=(n // bn,),
        in_specs=[
            pl.BlockSpec((bn, d), lambda i: (i, 0)),
            pl.BlockSpec((bn, d), lambda i: (i, 0)),
            pl.BlockSpec((d, d), lambda i: (0, 0)),
            pl.BlockSpec((d, e), lambda i: (0, 0)),
        ],
        out_specs=[
            pl.BlockSpec((bn, e), lambda i: (i, 0)),
            pl.BlockSpec((bn, topk), lambda i: (i, 0)),
            pl.BlockSpec((bn, topk), lambda i: (i, 0)),
            pl.BlockSpec((1, e), lambda i: (0, 0)),
            pl.BlockSpec((1, 1), lambda i: (0, 0)),
        ],
        out_shape=[
            jax.ShapeDtypeStruct((n, e), jnp.float32),
            jax.ShapeDtypeStruct((n, topk), jnp.int32),
            jax.ShapeDtypeStruct((n, topk), jnp.float32),
            jax.ShapeDtypeStruct((1, e), jnp.int32),
            jax.ShapeDtypeStruct((1, 1), jnp.float32),
        ],
        scratch_shapes=[pltpu.VMEM((1, e), jnp.float32)],
        compiler_params=pltpu.CompilerParams(
            dimension_semantics=("arbitrary",)),
    )(r, wkv, wo, wt)


def kernel(hidden_states, Wr, Wk, Wv, Wo, w_decay, u_bonus, router_W):
    b, s, d = hidden_states.shape
    e = router_W.shape[0]
    topk = 8
    n = b * s

    x = hidden_states.reshape(n, d)
    r = _matmul(x, Wr, act="sigmoid")
    k = _matmul(x, Wk)
    v = _matmul(x, Wv)

    wkv3 = _wkv_scan(k.reshape(b, s, d), v.reshape(b, s, d),
                     w_decay.reshape(1, d), u_bonus.reshape(1, d))
    wkv = wkv3.reshape(n, d)

    probs, idx, wts, cnt, loss = _router(r, wkv, Wo, router_W.T, topk)
    return (idx.reshape(b, s, topk),
            wts.reshape(b, s, topk),
            loss.reshape(()),
            probs.reshape(b, s, e),
            cnt.reshape(e))


# speedup vs baseline: 9.8942x; 1.0454x over previous
import functools

import jax
import jax.numpy as jnp
from jax.experimental import pallas as pl
from jax.experimental.pallas import tpu as pltpu


def _mm2_body(x_ref, wk_ref, wv_ref, k_ref, v_ref):
    xb = x_ref[...]
    k_ref[...] = jnp.dot(xb, wk_ref[...], preferred_element_type=jnp.float32)
    v_ref[...] = jnp.dot(xb, wv_ref[...], preferred_element_type=jnp.float32)


def _matmul_kv(x, wk, wv, bn=256):
    n, d = x.shape
    bn = min(bn, n)
    return pl.pallas_call(
        _mm2_body,
        grid=(n // bn,),
        in_specs=[
            pl.BlockSpec((bn, d), lambda i: (i, 0)),
            pl.BlockSpec((d, d), lambda i: (0, 0)),
            pl.BlockSpec((d, d), lambda i: (0, 0)),
        ],
        out_specs=[
            pl.BlockSpec((bn, d), lambda i: (i, 0)),
            pl.BlockSpec((bn, d), lambda i: (i, 0)),
        ],
        out_shape=[
            jax.ShapeDtypeStruct((n, d), jnp.float32),
            jax.ShapeDtypeStruct((n, d), jnp.float32),
        ],
        compiler_params=pltpu.CompilerParams(
            dimension_semantics=("arbitrary",)),
    )(x, wk, wv)


_T = 8


def _scan_body(w_ref, u_ref, k_ref, v_ref, o_ref,
               a_ref, b_ref, ek_ref, ekv_ref, eku_ref, euv_ref, *, sc):
    s = pl.program_id(0)

    @pl.when(s == 0)
    def _():
        a_ref[...] = jnp.zeros_like(a_ref)
        b_ref[...] = jnp.zeros_like(b_ref)

    ub8 = jnp.broadcast_to(u_ref[...][:, None, :], (1, _T, w_ref.shape[1]))

    def pre(g, _):
        kg = k_ref[:, pl.ds(g * _T, _T), :]
        vg = v_ref[:, pl.ds(g * _T, _T), :]
        ek = jnp.exp(jnp.clip(kg, -30.0, 30.0))
        eku = jnp.exp(jnp.clip(ub8 + kg, -30.0, 30.0))
        ek_ref[:, pl.ds(g * _T, _T), :] = ek
        ekv_ref[:, pl.ds(g * _T, _T), :] = ek * vg
        eku_ref[:, pl.ds(g * _T, _T), :] = eku
        euv_ref[:, pl.ds(g * _T, _T), :] = eku * vg
        return 0

    jax.lax.fori_loop(0, sc // _T, pre, 0, unroll=2)

    decay = jnp.exp(-w_ref[...])

    def step(j, carry):
        a, b = carry
        o_ref[:, j] = (a + euv_ref[:, j]) / ((b + eku_ref[:, j]) + 1e-8)
        a = decay * (a + ekv_ref[:, j])
        b = decay * (b + ek_ref[:, j])
        return (a, b)

    a1, b1 = jax.lax.fori_loop(
        0, sc, step, (a_ref[...], b_ref[...]), unroll=8)
    a_ref[...] = a1
    b_ref[...] = b1


def _wkv_scan(k3, v3, wd, ub, sc=128):
    b, s, d = k3.shape
    sc = min(sc, s)
    return pl.pallas_call(
        functools.partial(_scan_body, sc=sc),
        grid=(s // sc,),
        in_specs=[
            pl.BlockSpec((1, d), lambda i: (0, 0)),
            pl.BlockSpec((1, d), lambda i: (0, 0)),
            pl.BlockSpec((b, sc, d), lambda i: (0, i, 0)),
            pl.BlockSpec((b, sc, d), lambda i: (0, i, 0)),
        ],
        out_specs=pl.BlockSpec((b, sc, d), lambda i: (0, i, 0)),
        out_shape=jax.ShapeDtypeStruct((b, s, d), jnp.float32),
        scratch_shapes=[
            pltpu.VMEM((b, d), jnp.float32),
            pltpu.VMEM((b, d), jnp.float32),
            pltpu.VMEM((b, sc, d), jnp.float32),
            pltpu.VMEM((b, sc, d), jnp.float32),
            pltpu.VMEM((b, sc, d), jnp.float32),
            pltpu.VMEM((b, sc, d), jnp.float32),
        ],
        compiler_params=pltpu.CompilerParams(
            dimension_semantics=("arbitrary",)),
    )(wd, ub, k3, v3)


def _router_body(x_ref, wkv_ref, wr_ref, wo_ref, wt_ref,
                 probs_ref, idx_ref, wts_ref, cnt_ref, loss_ref,
                 acc_ref, *, bn, n_total, e, topk):
    i = pl.program_id(0)

    @pl.when(i == 0)
    def _():
        acc_ref[...] = jnp.zeros_like(acc_ref)
        cnt_ref[...] = jnp.zeros_like(cnt_ref)

    r = jax.nn.sigmoid(jnp.dot(x_ref[...], wr_ref[...],
                               preferred_element_type=jnp.float32))
    o = jnp.dot(r * wkv_ref[...], wo_ref[...],
                preferred_element_type=jnp.float32)
    logits = jnp.dot(o, wt_ref[...], preferred_element_type=jnp.float32)

    m = jnp.max(logits, axis=1, keepdims=True)
    p = jnp.exp(logits - m)
    probs = p / jnp.sum(p, axis=1, keepdims=True)
    probs_ref[...] = probs
    acc_ref[...] += jnp.sum(probs, axis=0, keepdims=True)

    iota = jax.lax.broadcasted_iota(jnp.int32, (bn, e), 1)
    work = probs
    vals, idxs = [], []
    onehot_sum = jnp.zeros((bn, e), jnp.float32)
    for _ in range(topk):
        mv = jnp.max(work, axis=1, keepdims=True)
        cand = jnp.where(work == mv, iota, e)
        mi = jnp.min(cand, axis=1, keepdims=True)
        sel = iota == mi
        vals.append(mv)
        idxs.append(mi)
        onehot_sum = onehot_sum + sel.astype(jnp.float32)
        work = jnp.where(sel, -1.0, work)
    cnt_ref[...] += jnp.sum(onehot_sum, axis=0,
                            keepdims=True).astype(jnp.int32)

    v8 = jnp.concatenate(vals, axis=1)
    idx_ref[...] = jnp.concatenate(idxs, axis=1).astype(jnp.int32)
    wts_ref[...] = v8 / (jnp.sum(v8, axis=1, keepdims=True) + 1e-8)

    @pl.when(i == pl.num_programs(0) - 1)
    def _():
        mean = acc_ref[...] / float(n_total)
        u = 1.0 / e
        kl = jnp.sum(u * (jnp.log(u) - jnp.log(mean + 1e-20)))
        loss_ref[...] = jnp.full((1, 1), 1.0 / e) * kl


def _router(x, wkv, wr, wo, wt, topk, bn=256):
    n, d = x.shape
    e = wt.shape[1]
    bn = min(bn, n)
    return pl.pallas_call(
        functools.partial(_router_body, bn=bn, n_total=n, e=e, topk=topk),
        grid=(n // bn,),
        in_specs=[
            pl.BlockSpec((bn, d), lambda i: (i, 0)),
            pl.BlockSpec((bn, d), lambda i: (i, 0)),
            pl.BlockSpec((d, d), lambda i: (0, 0)),
            pl.BlockSpec((d, d), lambda i: (0, 0)),
            pl.BlockSpec((d, e), lambda i: (0, 0)),
        ],
        out_specs=[
            pl.BlockSpec((bn, e), lambda i: (i, 0)),
            pl.BlockSpec((bn, topk), lambda i: (i, 0)),
            pl.BlockSpec((bn, topk), lambda i: (i, 0)),
            pl.BlockSpec((1, e), lambda i: (0, 0)),
            pl.BlockSpec((1, 1), lambda i: (0, 0)),
        ],
        out_shape=[
            jax.ShapeDtypeStruct((n, e), jnp.float32),
            jax.ShapeDtypeStruct((n, topk), jnp.int32),
            jax.ShapeDtypeStruct((n, topk), jnp.float32),
            jax.ShapeDtypeStruct((1, e), jnp.int32),
            jax.ShapeDtypeStruct((1, 1), jnp.float32),
        ],
        scratch_shapes=[pltpu.VMEM((1, e), jnp.float32)],
        compiler_params=pltpu.CompilerParams(
            dimension_semantics=("arbitrary",)),
    )(x, wkv, wr, wo, wt)


def kernel(hidden_states, Wr, Wk, Wv, Wo, w_decay, u_bonus, router_W):
    b, s, d = hidden_states.shape
    e = router_W.shape[0]
    topk = 8
    n = b * s

    x = hidden_states.reshape(n, d)
    k, v = _matmul_kv(x, Wk, Wv)

    wkv3 = _wkv_scan(k.reshape(b, s, d), v.reshape(b, s, d),
                     w_decay.reshape(1, d), u_bonus.reshape(1, d))
    wkv = wkv3.reshape(n, d)

    probs, idx, wts, cnt, loss = _router(x, wkv, Wr, Wo, router_W.T, topk)
    return (idx.reshape(b, s, topk),
            wts.reshape(b, s, topk),
            loss.reshape(()),
            probs.reshape(b, s, e),
            cnt.reshape(e))


# speedup vs baseline: 10.8476x; 1.0964x over previous
import functools

import jax
import jax.numpy as jnp
from jax.experimental import pallas as pl
from jax.experimental.pallas import tpu as pltpu


def _mm2_body(x_ref, wk_ref, wv_ref, k_ref, v_ref, *, dq):
    xb = x_ref[...]
    bn = xb.shape[0]
    k = jnp.dot(xb, wk_ref[...], preferred_element_type=jnp.float32)
    v = jnp.dot(xb, wv_ref[...], preferred_element_type=jnp.float32)
    k_ref[...] = k.reshape(bn, 8, dq)
    v_ref[...] = v.reshape(bn, 8, dq)


def _matmul_kv(x, wk, wv, b, s, bn=256):
    n, d = x.shape
    bn = min(bn, s)
    spb = s // bn
    dq = d // 8
    return pl.pallas_call(
        functools.partial(_mm2_body, dq=dq),
        grid=(b, spb),
        in_specs=[
            pl.BlockSpec((bn, d), lambda bi, st: (bi * spb + st, 0)),
            pl.BlockSpec((d, d), lambda bi, st: (0, 0)),
            pl.BlockSpec((d, d), lambda bi, st: (0, 0)),
        ],
        out_specs=[
            pl.BlockSpec((bn, 8, dq), lambda bi, st: (st, 0, bi)),
            pl.BlockSpec((bn, 8, dq), lambda bi, st: (st, 0, bi)),
        ],
        out_shape=[
            jax.ShapeDtypeStruct((s, 8, b * dq), jnp.float32),
            jax.ShapeDtypeStruct((s, 8, b * dq), jnp.float32),
        ],
        compiler_params=pltpu.CompilerParams(
            dimension_semantics=("arbitrary", "arbitrary")),
    )(x, wk, wv)


_T = 8


def _scan_body(w_ref, u_ref, k_ref, v_ref, o_ref,
               a_ref, b_ref, ek_ref, ekv_ref, eku_ref, euv_ref,
               *, sc, b, d):
    s = pl.program_id(0)
    dq = d // 8

    @pl.when(s == 0)
    def _():
        a_ref[...] = jnp.zeros_like(a_ref)
        b_ref[...] = jnp.zeros_like(b_ref)

    ub_s = jnp.tile(u_ref[...].reshape(8, dq), (1, b))[None]
    dec_s = jnp.exp(-jnp.tile(w_ref[...].reshape(8, dq), (1, b)))

    def pre(g, _):
        kg = k_ref[pl.ds(g * _T, _T)]
        vg = v_ref[pl.ds(g * _T, _T)]
        ek = jnp.exp(jnp.clip(kg, -30.0, 30.0))
        eku = jnp.exp(jnp.clip(ub_s + kg, -30.0, 30.0))
        ek_ref[pl.ds(g * _T, _T)] = ek
        ekv_ref[pl.ds(g * _T, _T)] = ek * vg
        eku_ref[pl.ds(g * _T, _T)] = eku
        euv_ref[pl.ds(g * _T, _T)] = eku * vg
        return 0

    jax.lax.fori_loop(0, sc // _T, pre, 0, unroll=2)

    def step(j, carry):
        a, bst = carry
        o_ref[j] = (a + euv_ref[j]) / ((bst + eku_ref[j]) + 1e-8)
        a = dec_s * (a + ekv_ref[j])
        bst = dec_s * (bst + ek_ref[j])
        return (a, bst)

    a1, b1 = jax.lax.fori_loop(
        0, sc, step, (a_ref[...], b_ref[...]), unroll=8)
    a_ref[...] = a1
    b_ref[...] = b1


def _wkv_scan(k4, v4, wd, ub, b, d, sc=128):
    s = k4.shape[0]
    sc = min(sc, s)
    bc = k4.shape[2]
    return pl.pallas_call(
        functools.partial(_scan_body, sc=sc, b=b, d=d),
        grid=(s // sc,),
        in_specs=[
            pl.BlockSpec((1, d), lambda i: (0, 0)),
            pl.BlockSpec((1, d), lambda i: (0, 0)),
            pl.BlockSpec((sc, 8, bc), lambda i: (i, 0, 0)),
            pl.BlockSpec((sc, 8, bc), lambda i: (i, 0, 0)),
        ],
        out_specs=pl.BlockSpec((sc, 8, bc), lambda i: (i, 0, 0)),
        out_shape=jax.ShapeDtypeStruct((s, 8, bc), jnp.float32),
        scratch_shapes=[
            pltpu.VMEM((8, bc), jnp.float32),
            pltpu.VMEM((8, bc), jnp.float32),
            pltpu.VMEM((sc, 8, bc), jnp.float32),
            pltpu.VMEM((sc, 8, bc), jnp.float32),
            pltpu.VMEM((sc, 8, bc), jnp.float32),
            pltpu.VMEM((sc, 8, bc), jnp.float32),
        ],
        compiler_params=pltpu.CompilerParams(
            dimension_semantics=("arbitrary",)),
    )(wd, ub, k4, v4)


def _router_body(x_ref, wkv_ref, wr_ref, wo_ref, wt_ref,
                 probs_ref, idx_ref, wts_ref, cnt_ref, loss_ref,
                 acc_ref, *, bn, n_total, e, topk):
    i = pl.program_id(0)

    @pl.when(i == 0)
    def _():
        acc_ref[...] = jnp.zeros_like(acc_ref)
        cnt_ref[...] = jnp.zeros_like(cnt_ref)

    r = jax.nn.sigmoid(jnp.dot(x_ref[...], wr_ref[...],
                               preferred_element_type=jnp.float32))
    wkv = wkv_ref[...].reshape(bn, wr_ref.shape[0])
    o = jnp.dot(r * wkv, wo_ref[...],
                preferred_element_type=jnp.float32)
    logits = jnp.dot(o, wt_ref[...], preferred_element_type=jnp.float32)

    m = jnp.max(logits, axis=1, keepdims=True)
    p = jnp.exp(logits - m)
    probs = p / jnp.sum(p, axis=1, keepdims=True)
    probs_ref[...] = probs
    acc_ref[...] += jnp.sum(probs, axis=0, keepdims=True)

    iota = jax.lax.broadcasted_iota(jnp.int32, (bn, e), 1)
    work = probs
    vals, idxs = [], []
    onehot_sum = jnp.zeros((bn, e), jnp.float32)
    for _ in range(topk):
        mv = jnp.max(work, axis=1, keepdims=True)
        cand = jnp.where(work == mv, iota, e)
        mi = jnp.min(cand, axis=1, keepdims=True)
        sel = iota == mi
        vals.append(mv)
        idxs.append(mi)
        onehot_sum = onehot_sum + sel.astype(jnp.float32)
        work = jnp.where(sel, -1.0, work)
    cnt_ref[...] += jnp.sum(onehot_sum, axis=0,
                            keepdims=True).astype(jnp.int32)

    v8 = jnp.concatenate(vals, axis=1)
    idx_ref[...] = jnp.concatenate(idxs, axis=1).astype(jnp.int32)
    wts_ref[...] = v8 / (jnp.sum(v8, axis=1, keepdims=True) + 1e-8)

    @pl.when(i == pl.num_programs(0) - 1)
    def _():
        mean = acc_ref[...] / float(n_total)
        u = 1.0 / e
        kl = jnp.sum(u * (jnp.log(u) - jnp.log(mean + 1e-20)))
        loss_ref[...] = jnp.full((1, 1), 1.0 / e) * kl


def _router(x, wkv, wr, wo, wt, topk, s, bn=256):
    n, d = x.shape
    e = wt.shape[1]
    bn = min(bn, s)
    spb = s // bn
    dq = d // 8
    return pl.pallas_call(
        functools.partial(_router_body, bn=bn, n_total=n, e=e, topk=topk),
        grid=(n // bn,),
        in_specs=[
            pl.BlockSpec((bn, d), lambda i: (i, 0)),
            pl.BlockSpec((bn, 8, dq), lambda i: (i % spb, 0, i // spb)),
            pl.BlockSpec((d, d), lambda i: (0, 0)),
            pl.BlockSpec((d, d), lambda i: (0, 0)),
            pl.BlockSpec((d, e), lambda i: (0, 0)),
        ],
        out_specs=[
            pl.BlockSpec((bn, e), lambda i: (i, 0)),
            pl.BlockSpec((bn, topk), lambda i: (i, 0)),
            pl.BlockSpec((bn, topk), lambda i: (i, 0)),
            pl.BlockSpec((1, e), lambda i: (0, 0)),
            pl.BlockSpec((1, 1), lambda i: (0, 0)),
        ],
        out_shape=[
            jax.ShapeDtypeStruct((n, e), jnp.float32),
            jax.ShapeDtypeStruct((n, topk), jnp.int32),
            jax.ShapeDtypeStruct((n, topk), jnp.float32),
            jax.ShapeDtypeStruct((1, e), jnp.int32),
            jax.ShapeDtypeStruct((1, 1), jnp.float32),
        ],
        scratch_shapes=[pltpu.VMEM((1, e), jnp.float32)],
        compiler_params=pltpu.CompilerParams(
            dimension_semantics=("arbitrary",)),
    )(x, wkv, wr, wo, wt)


def kernel(hidden_states, Wr, Wk, Wv, Wo, w_decay, u_bonus, router_W):
    b, s, d = hidden_states.shape
    e = router_W.shape[0]
    topk = 8
    n = b * s

    x = hidden_states.reshape(n, d)
    k4, v4 = _matmul_kv(x, Wk, Wv, b, s)

    wkv4 = _wkv_scan(k4, v4, w_decay.reshape(1, d), u_bonus.reshape(1, d),
                     b, d)

    probs, idx, wts, cnt, loss = _router(x, wkv4, Wr, Wo, router_W.T,
                                         topk, s)
    return (idx.reshape(b, s, topk),
            wts.reshape(b, s, topk),
            loss.reshape(()),
            probs.reshape(b, s, e),
            cnt.reshape(e))
